# baseline (device time: 16002 ns/iter reference)
import os

import jax
import jax.numpy as jnp
from jax import lax
from jax.experimental import pallas as pl
from jax.experimental.pallas import tpu as pltpu

N_DEV = 32

_VARIANT = os.environ.get("KERNEL_VARIANT", "full")
_USE_BARRIER = _VARIANT in ("full", "barrier_only")
_USE_A2A = _VARIANT in ("full", "nobarrier", "a2a_only") or _VARIANT.startswith(
    "a2a_h"
)
_USE_GEMM = not _VARIANT.startswith("a2a")
_N_HOPS = int(_VARIANT[5:]) if _VARIANT.startswith("a2a_h") else N_DEV - 1


def _gelu(y):
    c = 0.7978845608028654
    return 0.5 * y * (1.0 + jnp.tanh(c * (y + 0.044715 * y ** 3)))


def _kernel_grid(x, w_mat):
    m_per, k = x.shape
    n = w_mat.shape[1]
    n_per = n // N_DEV
    G = 8
    n_blk = n // G
    cpg = n_blk // n_per

    def row_slice(ref, idx):
        return ref.at[pl.ds(pl.multiple_of(idx * m_per, m_per), m_per), :]

    def body(x_ref, w_ref, out_ref, sbuf_ref, send_sems, recv_sems, local_sem):
        g = pl.program_id(0)
        me = lax.axis_index("i")
        barrier_sem = pltpu.get_barrier_semaphore()

        @pl.when(g == 0)
        def _():
            for h in range(1, N_DEV):
                pl.semaphore_signal(
                    barrier_sem,
                    inc=1,
                    device_id=(lax.rem(me + h, N_DEV),),
                    device_id_type=pl.DeviceIdType.MESH,
                )

        y_blk = jnp.dot(
            x_ref[...], w_ref[...], preferred_element_type=jnp.float32
        )
        y_blk = _gelu(y_blk)
        base = pl.multiple_of(g * cpg, cpg)
        for j in range(cpg):
            sbuf_ref[base + j] = y_blk[:, j * n_per:(j + 1) * n_per]

        @pl.when(g == G - 1)
        def _():
            pl.semaphore_wait(barrier_sem, N_DEV - 1)
            sends = []
            for h in range(1, N_DEV):
                t = lax.rem(me + h, N_DEV)
                rdma = pltpu.make_async_remote_copy(
                    src_ref=sbuf_ref.at[t],
                    dst_ref=row_slice(out_ref, me),
                    send_sem=send_sems.at[h],
                    recv_sem=recv_sems.at[h],
                    device_id=(t,),
                    device_id_type=pl.DeviceIdType.MESH,
                )
                rdma.start()
                sends.append(rdma)

            local = pltpu.make_async_copy(
                sbuf_ref.at[me], row_slice(out_ref, me), local_sem
            )
            local.start()

            for h in range(1, N_DEV):
                s = lax.rem(me - h + N_DEV, N_DEV)
                recv = pltpu.make_async_remote_copy(
                    src_ref=sbuf_ref.at[0],
                    dst_ref=row_slice(out_ref, s),
                    send_sem=send_sems.at[h],
                    recv_sem=recv_sems.at[h],
                    device_id=(s,),
                    device_id_type=pl.DeviceIdType.MESH,
                )
                recv.wait_recv()

            local.wait()
            for rdma in sends:
                rdma.wait_send()

    return pl.pallas_call(
        body,
        grid=(G,),
        out_shape=jax.ShapeDtypeStruct((N_DEV * m_per, n_per), jnp.float32),
        in_specs=[
            pl.BlockSpec(memory_space=pltpu.VMEM),
            pl.BlockSpec((k, n_blk), lambda g: (0, g)),
        ],
        out_specs=pl.BlockSpec(memory_space=pltpu.VMEM),
        scratch_shapes=[
            pltpu.VMEM((N_DEV, m_per, n_per), jnp.float32),
            pltpu.SemaphoreType.DMA((N_DEV,)),
            pltpu.SemaphoreType.DMA((N_DEV,)),
            pltpu.SemaphoreType.DMA,
        ],
        compiler_params=pltpu.CompilerParams(
            collective_id=0,
            dimension_semantics=("arbitrary",),
        ),
    )(x, w_mat)


def _kernel_flat(x, w_mat):
    m_per, k = x.shape
    n = w_mat.shape[1]
    n_per = n // N_DEV

    def row_slice(ref, idx):
        return ref.at[pl.ds(pl.multiple_of(idx * m_per, m_per), m_per), :]

    def body(x_ref, w_ref, out_ref, sbuf_ref, send_sems, recv_sems, local_sem):
        me = lax.axis_index("i")

        if _USE_BARRIER:
            barrier_sem = pltpu.get_barrier_semaphore()
            for h in range(1, N_DEV):
                pl.semaphore_signal(
                    barrier_sem,
                    inc=1,
                    device_id=(lax.rem(me + h, N_DEV),),
                    device_id_type=pl.DeviceIdType.MESH,
                )

        if _USE_GEMM:
            y = jnp.dot(
                x_ref[...], w_ref[...], preferred_element_type=jnp.float32
            )
            y = _gelu(y)
            n_extract = 1 if _VARIANT == "gemm_only" else N_DEV
            for t in range(n_extract):
                sbuf_ref[t] = y[:, t * n_per:(t + 1) * n_per]
        else:
            for t in range(N_DEV):
                sbuf_ref[t] = x_ref[:, 0:n_per]

        if _VARIANT == "gemm_only":
            local = pltpu.make_async_copy(
                sbuf_ref.at[0], row_slice(out_ref, me), local_sem
            )
            local.start()
            local.wait()
            return

        if _USE_BARRIER:
            pl.semaphore_wait(barrier_sem, N_DEV - 1)

        if _VARIANT in ("compute_only", "barrier_only"):
            local = pltpu.make_async_copy(
                sbuf_ref.at[me], row_slice(out_ref, me), local_sem
            )
            local.start()
            local.wait()
            return

        sends = []
        for h in range(1, _N_HOPS + 1):
            t = lax.rem(me + h, N_DEV)
            rdma = pltpu.make_async_remote_copy(
                src_ref=sbuf_ref.at[t],
                dst_ref=row_slice(out_ref, me),
                send_sem=send_sems.at[h],
                recv_sem=recv_sems.at[h],
                device_id=(t,),
                device_id_type=pl.DeviceIdType.MESH,
            )
            rdma.start()
            sends.append(rdma)

        local = pltpu.make_async_copy(
            sbuf_ref.at[me], row_slice(out_ref, me), local_sem
        )
        local.start()

        for h in range(1, _N_HOPS + 1):
            s = lax.rem(me - h + N_DEV, N_DEV)
            recv = pltpu.make_async_remote_copy(
                src_ref=sbuf_ref.at[0],
                dst_ref=row_slice(out_ref, s),
                send_sem=send_sems.at[h],
                recv_sem=recv_sems.at[h],
                device_id=(s,),
                device_id_type=pl.DeviceIdType.MESH,
            )
            recv.wait_recv()

        local.wait()
        for rdma in sends:
            rdma.wait_send()

    return pl.pallas_call(
        body,
        out_shape=jax.ShapeDtypeStruct((N_DEV * m_per, n_per), jnp.float32),
        in_specs=[
            pl.BlockSpec(memory_space=pltpu.VMEM),
            pl.BlockSpec(memory_space=pltpu.VMEM),
        ],
        out_specs=pl.BlockSpec(memory_space=pltpu.VMEM),
        scratch_shapes=[
            pltpu.VMEM((N_DEV, m_per, n_per), jnp.float32),
            pltpu.SemaphoreType.DMA((N_DEV,)),
            pltpu.SemaphoreType.DMA((N_DEV,)),
            pltpu.SemaphoreType.DMA,
        ],
        compiler_params=(
            pltpu.CompilerParams(collective_id=0) if _USE_BARRIER else None
        ),
    )(x, w_mat)


def kernel(x, w_mat):
    if _VARIANT == "grid":
        return _kernel_grid(x, w_mat)
    return _kernel_flat(x, w_mat)
